# baseline (speedup 1.0000x reference)
import functools

import jax
import jax.numpy as jnp
from jax import lax
from jax.experimental import pallas as pl
from jax.experimental.pallas import tpu as pltpu
from jax.experimental.pallas import tpu_sc as plsc

_IDX_CHUNK = 64


def kernel(labels, table):
    B, = labels.shape
    V, D = table.shape
    info = plsc.get_sparse_core_info()
    NC, NS = info.num_cores, info.num_subcores
    NW = NC * NS
    b_per_w = B // NW
    n_chunks = b_per_w // _IDX_CHUNK

    mesh = plsc.VectorSubcoreMesh(core_axis_name="c", subcore_axis_name="s")

    @functools.partial(
        pl.kernel,
        mesh=mesh,
        out_type=jax.ShapeDtypeStruct((B, D), jnp.float32),
        compiler_params=pltpu.CompilerParams(needs_layout_passes=False),
        scratch_types=[
            pltpu.VMEM_SHARED((V, D), jnp.float32),
            pltpu.VMEM((n_chunks, _IDX_CHUNK), jnp.int32),
            pltpu.VMEM((b_per_w, D), jnp.float32),
            pltpu.SemaphoreType.DMA,
            pltpu.SemaphoreType.DMA,
        ],
    )
    def _emb(labels_hbm, table_hbm, out_hbm, table_s, idx_v, rows_v, gsem, osem):
        sid = lax.axis_index("s")
        wid = sid * NC + lax.axis_index("c")
        base = wid * b_per_w

        @pl.when(sid == 0)
        def _stage():
            pltpu.sync_copy(table_hbm, table_s)

        pltpu.sync_copy(labels_hbm.at[wid], idx_v)
        plsc.subcore_barrier()

        gathers = [
            pltpu.async_copy(
                table_s.at[idx_v.at[c]],
                rows_v.at[pl.ds(c * _IDX_CHUNK, _IDX_CHUNK)],
                gsem,
            )
            for c in range(n_chunks)
        ]
        writes = []
        for c in range(n_chunks):
            gathers[c].wait()
            writes.append(
                pltpu.async_copy(
                    rows_v.at[pl.ds(c * _IDX_CHUNK, _IDX_CHUNK)],
                    out_hbm.at[pl.ds(base + c * _IDX_CHUNK, _IDX_CHUNK)],
                    osem,
                )
            )
        for w in writes:
            w.wait()

    labels_r = labels.reshape(NW, n_chunks, _IDX_CHUNK).astype(jnp.int32)
    return _emb(labels_r, table)


# speedup vs baseline: 1.0078x; 1.0078x over previous
import functools

import jax
import jax.numpy as jnp
from jax import lax
from jax.experimental import pallas as pl
from jax.experimental.pallas import tpu as pltpu
from jax.experimental.pallas import tpu_sc as plsc

_IDX_CHUNK = 128


def kernel(labels, table):
    B, = labels.shape
    V, D = table.shape
    info = plsc.get_sparse_core_info()
    NC, NS = info.num_cores, info.num_subcores
    NW = NC * NS
    b_per_w = B // NW
    n_chunks = b_per_w // _IDX_CHUNK

    mesh = plsc.VectorSubcoreMesh(core_axis_name="c", subcore_axis_name="s")

    @functools.partial(
        pl.kernel,
        mesh=mesh,
        out_type=jax.ShapeDtypeStruct((B, D), jnp.float32),
        compiler_params=pltpu.CompilerParams(needs_layout_passes=False),
        scratch_types=[
            pltpu.VMEM_SHARED((V, D), jnp.float32),
            pltpu.VMEM((n_chunks, _IDX_CHUNK), jnp.int32),
            pltpu.VMEM((b_per_w, D), jnp.float32),
            pltpu.SemaphoreType.DMA,
            pltpu.SemaphoreType.DMA,
        ],
    )
    def _emb(labels_hbm, table_hbm, out_hbm, table_s, idx_v, rows_v, gsem, osem):
        sid = lax.axis_index("s")
        wid = sid * NC + lax.axis_index("c")
        base = wid * b_per_w

        @pl.when(sid == 0)
        def _stage():
            pltpu.sync_copy(table_hbm, table_s)

        pltpu.sync_copy(labels_hbm.at[wid], idx_v)
        plsc.subcore_barrier()

        gathers = [
            pltpu.async_copy(
                table_s.at[idx_v.at[c]],
                rows_v.at[pl.ds(c * _IDX_CHUNK, _IDX_CHUNK)],
                gsem,
            )
            for c in range(n_chunks)
        ]
        writes = []
        for c in range(n_chunks):
            gathers[c].wait()
            writes.append(
                pltpu.async_copy(
                    rows_v.at[pl.ds(c * _IDX_CHUNK, _IDX_CHUNK)],
                    out_hbm.at[pl.ds(base + c * _IDX_CHUNK, _IDX_CHUNK)],
                    osem,
                )
            )
        for w in writes:
            w.wait()

    labels_r = labels.reshape(NW, n_chunks, _IDX_CHUNK).astype(jnp.int32)
    return _emb(labels_r, table)
